# baseline (speedup 1.0000x reference)
import functools

import jax
import jax.numpy as jnp
from jax import lax
from jax.experimental import pallas as pl
from jax.experimental.pallas import tpu as pltpu
from jax.experimental.pallas import tpu_sc as plsc

_MEM_SIZE = 100000
_DIM = 16
_NQ = 1024
_THRESHOLD = 0.8

_BLK = 2048
_NSTEPS = -(-_MEM_SIZE // _BLK)

_NC = 2
_NS = 16
_NW = _NC * _NS
_B_PER_W = _NQ // _NW


def _scan_body(x_ref, mem_ref, idx_ref, mask_ref, xn_ref, rmax_ref, ridx_ref):
    i = pl.program_id(0)

    @pl.when(i == 0)
    def _init():
        xv = x_ref[...]
        n = jnp.sqrt(jnp.sum(xv * xv, axis=1, keepdims=True))
        xn_ref[...] = xv / jnp.maximum(n, 1e-12)
        rmax_ref[...] = jnp.full((_NQ,), -jnp.inf, jnp.float32)
        ridx_ref[...] = jnp.zeros((_NQ,), jnp.int32)

    mem = mem_ref[...]
    n = jnp.sqrt(jnp.sum(mem * mem, axis=1, keepdims=True))
    mn = mem / jnp.maximum(n, 1e-12)
    sim = lax.dot_general(
        xn_ref[...].astype(jnp.bfloat16),
        mn.astype(jnp.bfloat16),
        (((1,), (1,)), ((), ())),
        preferred_element_type=jnp.float32,
    )
    gcol = lax.broadcasted_iota(jnp.int32, (_NQ, _BLK), 1) + i * _BLK
    sim = jnp.where(gcol < _MEM_SIZE, sim, -jnp.inf)
    lmax = jnp.max(sim, axis=1)
    cand = jnp.where(sim == lmax[:, None], gcol, jnp.iinfo(jnp.int32).max)
    lidx = jnp.min(cand, axis=1)
    better = lmax > rmax_ref[...]
    rmax_ref[...] = jnp.where(better, lmax, rmax_ref[...])
    ridx_ref[...] = jnp.where(better, lidx, ridx_ref[...])

    @pl.when(i == _NSTEPS - 1)
    def _fin():
        idx_ref[...] = ridx_ref[...]
        m = (rmax_ref[...] > _THRESHOLD).astype(jnp.float32)
        mask_ref[...] = jnp.broadcast_to(m[:, None], (_NQ, _DIM))


_scan = pl.pallas_call(
    _scan_body,
    grid=(_NSTEPS,),
    in_specs=[
        pl.BlockSpec((_NQ, _DIM), lambda i: (0, 0)),
        pl.BlockSpec((_BLK, _DIM), lambda i: (i, 0)),
    ],
    out_specs=[
        pl.BlockSpec((_NQ,), lambda i: (0,)),
        pl.BlockSpec((_NQ, _DIM), lambda i: (0, 0)),
    ],
    out_shape=[
        jax.ShapeDtypeStruct((_NQ,), jnp.int32),
        jax.ShapeDtypeStruct((_NQ, _DIM), jnp.float32),
    ],
    scratch_shapes=[
        pltpu.VMEM((_NQ, _DIM), jnp.float32),
        pltpu.VMEM((_NQ,), jnp.float32),
        pltpu.VMEM((_NQ,), jnp.int32),
    ],
)


@functools.partial(
    pl.kernel,
    out_type=jax.ShapeDtypeStruct((_NQ, _DIM), jnp.float32),
    mesh=plsc.VectorSubcoreMesh(
        core_axis_name="c", subcore_axis_name="s", num_cores=_NC, num_subcores=_NS
    ),
    scratch_types=[
        pltpu.VMEM((_B_PER_W,), jnp.int32),
        pltpu.VMEM((_B_PER_W, _DIM), jnp.float32),
        pltpu.VMEM((_B_PER_W, _DIM), jnp.float32),
        pltpu.SemaphoreType.DMA,
    ],
    compiler_params=pltpu.CompilerParams(use_tc_tiling_on_sc=False),
)
def _gather(mem_hbm, idx_hbm, mask_hbm, out_hbm, idx_v, rows_v, mask_v, sem):
    wid = lax.axis_index("s") * _NC + lax.axis_index("c")
    base = wid * _B_PER_W
    pltpu.sync_copy(idx_hbm.at[pl.ds(base, _B_PER_W)], idx_v)
    pltpu.async_copy(mem_hbm.at[idx_v], rows_v, sem).wait()
    pltpu.sync_copy(mask_hbm.at[pl.ds(base, _B_PER_W)], mask_v)
    for j in range(_B_PER_W):
        rows_v[j] = rows_v[j] * mask_v[j]
    pltpu.sync_copy(rows_v, out_hbm.at[pl.ds(base, _B_PER_W)])


def kernel(x, memory):
    idx, maskf = _scan(x, memory)
    return _gather(memory, idx, maskf)


# speedup vs baseline: 1.3071x; 1.3071x over previous
import functools

import jax
import jax.numpy as jnp
from jax import lax
from jax.experimental import pallas as pl
from jax.experimental.pallas import tpu as pltpu
from jax.experimental.pallas import tpu_sc as plsc

_MEM_SIZE = 100000
_DIM = 16
_NQ = 1024
_THRESHOLD = 0.8

_BLK = 2048
_NSTEPS = -(-_MEM_SIZE // _BLK)

_NC = 2
_NS = 16
_NW = _NC * _NS
_B_PER_W = _NQ // _NW


def _scan_body(x_ref, memt_ref, idx_ref, mval_ref,
               xnb_ref, cols_ref, rmax_ref, ridx_ref, lidx_ref):
    i = pl.program_id(0)

    @pl.when(i == 0)
    def _init():
        xv = x_ref[...]
        n = jnp.sqrt(jnp.sum(xv * xv, axis=1, keepdims=True))
        xn = xv / jnp.maximum(n, 1e-12)
        xnb_ref[...] = xn.astype(jnp.bfloat16)
        iota = lax.broadcasted_iota(jnp.int32, (_BLK, 3), 0)
        csel = lax.broadcasted_iota(jnp.int32, (_BLK, 3), 1)
        colv = jnp.where(
            csel == 0,
            jnp.right_shift(iota, 3),
            jnp.where(csel == 1, jnp.bitwise_and(iota, 7), 1),
        )
        cols_ref[...] = colv.astype(jnp.bfloat16)
        rmax_ref[...] = jnp.full((_NQ, 1), -jnp.inf, jnp.float32)
        ridx_ref[...] = jnp.zeros((_NQ, 1), jnp.float32)

    memr = memt_ref[...]
    colid = lax.broadcasted_iota(jnp.int32, (1, _BLK), 1)
    valid = (colid + i * _BLK) < _MEM_SIZE
    memz = jnp.where(valid, memr, 0.0)
    sumd = jnp.sum(memz * memz, axis=0, keepdims=True)
    normf = jnp.sqrt(sumd)
    mn = memz / jnp.maximum(normf, 1e-12)
    sim = lax.dot_general(
        xnb_ref[...],
        mn.astype(jnp.bfloat16),
        (((1,), (0,)), ((), ())),
        preferred_element_type=jnp.float32,
    )
    lmax = jnp.max(sim, axis=1, keepdims=True)
    eqf = (sim == lmax).astype(jnp.bfloat16)
    aux = lax.dot_general(
        eqf,
        cols_ref[...],
        (((1,), (0,)), ((), ())),
        preferred_element_type=jnp.float32,
    )
    idxf = aux[:, 0:1] * 8.0 + aux[:, 1:2]
    cnt = aux[:, 2:3]
    lidx_ref[...] = idxf

    @pl.when(jnp.any(cnt != 1.0))
    def _tie_fallback():
        iota2 = lax.broadcasted_iota(jnp.int32, (_NQ, _BLK), 1).astype(jnp.float32)
        cand = jnp.where(sim == lmax, iota2, jnp.float32(2**24))
        first = jnp.min(cand, axis=1, keepdims=True)
        lidx_ref[...] = jnp.where(cnt == 1.0, idxf, first)

    glob = lidx_ref[...] + jnp.float32(i * _BLK)
    better = lmax > rmax_ref[...]
    rmax_ref[...] = jnp.where(better, lmax, rmax_ref[...])
    ridx_ref[...] = jnp.where(better, glob, ridx_ref[...])

    @pl.when(i == _NSTEPS - 1)
    def _fin():
        ri = jnp.minimum(ridx_ref[...], jnp.float32(_MEM_SIZE - 1))
        idx_ref[...] = jnp.reshape(ri.astype(jnp.int32), (_NQ,))
        mv = (rmax_ref[...] > _THRESHOLD).astype(jnp.float32)
        mval_ref[...] = jnp.reshape(mv, (_NQ,))


_scan = pl.pallas_call(
    _scan_body,
    grid=(_NSTEPS,),
    in_specs=[
        pl.BlockSpec((_NQ, _DIM), lambda i: (0, 0)),
        pl.BlockSpec((_DIM, _BLK), lambda i: (0, i)),
    ],
    out_specs=[
        pl.BlockSpec((_NQ,), lambda i: (0,)),
        pl.BlockSpec((_NQ,), lambda i: (0,)),
    ],
    out_shape=[
        jax.ShapeDtypeStruct((_NQ,), jnp.int32),
        jax.ShapeDtypeStruct((_NQ,), jnp.float32),
    ],
    scratch_shapes=[
        pltpu.VMEM((_NQ, _DIM), jnp.bfloat16),
        pltpu.VMEM((_BLK, 3), jnp.bfloat16),
        pltpu.VMEM((_NQ, 1), jnp.float32),
        pltpu.VMEM((_NQ, 1), jnp.float32),
        pltpu.VMEM((_NQ, 1), jnp.float32),
    ],
)


@functools.partial(
    pl.kernel,
    out_type=jax.ShapeDtypeStruct((_DIM, _NQ), jnp.float32),
    mesh=plsc.VectorSubcoreMesh(
        core_axis_name="c", subcore_axis_name="s", num_cores=_NC, num_subcores=_NS
    ),
    scratch_types=[
        pltpu.VMEM((_B_PER_W,), jnp.int32),
        pltpu.VMEM((_B_PER_W,), jnp.float32),
        pltpu.VMEM((_DIM, _B_PER_W), jnp.float32),
        pltpu.SemaphoreType.DMA,
    ],
    compiler_params=pltpu.CompilerParams(use_tc_tiling_on_sc=False),
)
def _gather(memt_hbm, idx_hbm, mval_hbm, out_hbm, idx_v, mask_v, cols_v, sem):
    wid = lax.axis_index("s") * _NC + lax.axis_index("c")
    base = wid * _B_PER_W
    pltpu.sync_copy(idx_hbm.at[pl.ds(base, _B_PER_W)], idx_v)
    pltpu.sync_copy(mval_hbm.at[pl.ds(base, _B_PER_W)], mask_v)
    descs = [
        pltpu.async_copy(memt_hbm.at[d].at[idx_v], cols_v.at[d], sem)
        for d in range(_DIM)
    ]
    for desc in descs:
        desc.wait()
    for d in range(_DIM):
        for c in range(_B_PER_W // 16):
            s = pl.ds(16 * c, 16)
            cols_v[d, s] = cols_v[d, s] * mask_v[s]
    pltpu.sync_copy(cols_v, out_hbm.at[:, pl.ds(base, _B_PER_W)])


def kernel(x, memory):
    idx, mval = _scan(x, memory.T)
    out_t = _gather(memory.T, idx, mval)
    return out_t.T
